# baseline (speedup 1.0000x reference)
import functools

import jax
import jax.numpy as jnp
from jax import lax
from jax.experimental import pallas as pl
from jax.experimental.pallas import tpu as pltpu
from jax.experimental.pallas import tpu_sc as plsc

_H = 512
_W = 512
_C = 3
_PS = 16
_N = 16
_P = 1024

_L = 16
_NC_COMBOS = _N * _C
_ROWS_PER_P = _NC_COMBOS * _PS
_PAIRS_PER_P = 2 * _ROWS_PER_P
_IDX_MINOR = 128
_IDX_MAJOR = _PAIRS_PER_P // _IDX_MINOR
_OUT_PER_P = _ROWS_PER_P * _PS
_CHUNKS_PER_IMROW = _W // _L
_TABLE_ROWS = _N * _C * _H * _CHUNKS_PER_IMROW

_NUM_WORKERS = 32
_P_PER_WORKER = _P // _NUM_WORKERS

_mesh = plsc.VectorSubcoreMesh(core_axis_name="c", subcore_axis_name="s")

_SRC_ROWS = _TABLE_ROWS * _L // 128


@functools.partial(
    pl.kernel,
    mesh=_mesh,
    out_type=jax.ShapeDtypeStruct((_TABLE_ROWS, _L), jnp.float32),
    compiler_params=pltpu.CompilerParams(
        needs_layout_passes=False, use_tc_tiling_on_sc=False
    ),
    scratch_types=[
        pltpu.VMEM((384, 128), jnp.float32),
        pltpu.VMEM((3072, _L), jnp.float32),
        pltpu.SemaphoreType.DMA,
    ],
)
def _sc_reformat(src, out, a_v, b_v, sem):
    wid = lax.axis_index("s") * 2 + lax.axis_index("c")
    srows = _SRC_ROWS // _NUM_WORKERS
    for chunk in range(8):
        base = wid * srows + chunk * 384
        pltpu.async_copy(src.at[pl.ds(base, 384)], a_v, sem).wait()

        def body(r, carry):
            for j in range(8):
                b_v[8 * r + j] = a_v[r, pl.ds(j * _L, _L)]
            return carry

        lax.fori_loop(0, 384, body, 0, unroll=4)
        pltpu.sync_copy(b_v, out.at[pl.ds(base * 8, 3072)])


@functools.partial(
    pl.kernel,
    mesh=_mesh,
    out_type=jax.ShapeDtypeStruct((_P * _ROWS_PER_P, 128), jnp.float32),
    compiler_params=pltpu.CompilerParams(
        needs_layout_passes=False, use_tc_tiling_on_sc=False
    ),
    scratch_types=[
        pltpu.VMEM((_P_PER_WORKER * _L,), jnp.int32),
        pltpu.VMEM((_P_PER_WORKER * _L,), jnp.int32),
        pltpu.VMEM((_IDX_MAJOR, _IDX_MINOR), jnp.int32),
        pltpu.VMEM((_PAIRS_PER_P, _L), jnp.float32),
        pltpu.VMEM((_ROWS_PER_P, _PS), jnp.float32),
        pltpu.SemaphoreType.DMA,
    ],
)
def _sc_gather(table, hpos, wpos, out, h_v, w_v, idx_v, pairs_v, stage_v, sem):
    wid = lax.axis_index("s") * 2 + lax.axis_index("c")
    sl = pl.ds(wid * (_P_PER_WORKER * _L), _P_PER_WORKER * _L)
    pltpu.sync_copy(hpos.at[sl], h_v)
    pltpu.sync_copy(wpos.at[sl], w_v)
    iota = lax.iota(jnp.int32, _L)

    def per_projection(k, carry):
        p = wid * _P_PER_WORKER + k
        hv = h_v[pl.ds(k * _L, _L)]
        wv = w_v[pl.ds(k * _L, _L)]
        hh = (hv + iota) & (_H - 1)
        c0 = wv >> 4
        rowbase = hh * _CHUNKS_PER_IMROW + c0
        delta = jnp.where(c0 == _CHUNKS_PER_IMROW - 1,
                          1 - _CHUNKS_PER_IMROW, 1)

        for nc in range(_NC_COMBOS):
            r0 = rowbase + nc * (_H * _CHUNKS_PER_IMROW)
            r1 = r0 + delta
            row = jnp.full((_L,), nc >> 2, jnp.int32)
            col = (nc & 3) * 32 + 2 * iota
            plsc.store_scatter(idx_v, [row, col], r0)
            plsc.store_scatter(idx_v, [row, col + 1], r1)

        copies = [
            pltpu.async_copy(
                table.at[idx_v.at[b]],
                pairs_v.at[pl.ds(b * _IDX_MINOR, _IDX_MINOR)],
                sem,
            )
            for b in range(_IDX_MAJOR)
        ]
        for cp in copies:
            cp.wait()

        lane_off = (wv & (_L - 1)) + iota
        roff = lane_off >> 4
        coff = lane_off & (_L - 1)

        def per_row(t, c2):
            g = plsc.load_gather(pairs_v, [2 * t + roff, coff])
            stage_v[t] = g
            return c2

        lax.fori_loop(0, _ROWS_PER_P, per_row, 0, unroll=8)
        pltpu.sync_copy(
            stage_v,
            out.at[pl.ds(p * _ROWS_PER_P, _ROWS_PER_P), pl.ds(0, _PS)],
        )
        return carry

    lax.fori_loop(0, _P_PER_WORKER, per_projection, 0)


_BP = 16


def _inds_body(h_sref, w_sref, o_ref):
    pid = pl.program_id(0)
    shape = (1, _N, _C, _PS, _PS)
    nn = lax.broadcasted_iota(jnp.int32, shape, 1)
    cc = lax.broadcasted_iota(jnp.int32, shape, 2)
    ii = lax.broadcasted_iota(jnp.int32, shape, 3)
    jj = lax.broadcasted_iota(jnp.int32, shape, 4)
    base = _W * _H * cc + _W * _H * _C * nn
    for b in range(_BP):
        h = h_sref[pid * _BP + b]
        w = w_sref[pid * _BP + b]
        o_ref[pl.ds(b, 1)] = (((w + jj) & (_W - 1))
                              + _W * ((h + ii) & (_H - 1)) + base)


def _tc_inds(h, w):
    grid_spec = pltpu.PrefetchScalarGridSpec(
        num_scalar_prefetch=2,
        grid=(_P // _BP,),
        in_specs=[],
        out_specs=pl.BlockSpec(
            (_BP, _N, _C, _PS, _PS), lambda i, h, w: (i, 0, 0, 0, 0)
        ),
    )
    return pl.pallas_call(
        _inds_body,
        grid_spec=grid_spec,
        out_shape=jax.ShapeDtypeStruct((_P, _N, _C, _PS, _PS), jnp.int32),
    )(h, w)


def kernel(imgs, position_inds_height, position_inds_width):
    h = position_inds_height.astype(jnp.int32)
    w = position_inds_width.astype(jnp.int32)
    table = _sc_reformat(imgs.reshape(_SRC_ROWS, 128))
    h_rep = jnp.broadcast_to(h[:, None], (_P, _L)).reshape(-1)
    w_rep = jnp.broadcast_to(w[:, None], (_P, _L)).reshape(-1)
    patches = (
        _sc_gather(table, h_rep, w_rep)[:, :_PS]
        .reshape(_P, _N, _C, _PS, _PS)
    )
    linear_inds = _tc_inds(h, w).reshape(-1)
    return patches, linear_inds


# speedup vs baseline: 1.3712x; 1.3712x over previous
import functools

import jax
import jax.numpy as jnp
from jax import lax
from jax.experimental import pallas as pl
from jax.experimental.pallas import tpu as pltpu
from jax.experimental.pallas import tpu_sc as plsc

_H = 512
_W = 512
_C = 3
_PS = 16
_N = 16
_P = 1024

_L = 16
_NC_COMBOS = _N * _C
_ROWS_PER_P = _NC_COMBOS * _PS
_PAIRS_PER_P = 2 * _ROWS_PER_P
_IDX_MINOR = 128
_IDX_MAJOR = _PAIRS_PER_P // _IDX_MINOR
_OUT_PER_P = _ROWS_PER_P * _PS
_CHUNKS_PER_IMROW = _W // _L
_TABLE_ROWS = _N * _C * _H * _CHUNKS_PER_IMROW

_NUM_WORKERS = 32
_P_PER_WORKER = _P // _NUM_WORKERS

_mesh = plsc.VectorSubcoreMesh(core_axis_name="c", subcore_axis_name="s")

@functools.partial(
    pl.kernel,
    mesh=_mesh,
    out_type=jax.ShapeDtypeStruct((_P * _ROWS_PER_P, 128), jnp.float32),
    compiler_params=pltpu.CompilerParams(
        needs_layout_passes=False, use_tc_tiling_on_sc=False
    ),
    scratch_types=[
        pltpu.VMEM((_P_PER_WORKER * _L,), jnp.int32),
        pltpu.VMEM((_P_PER_WORKER * _L,), jnp.int32),
        pltpu.VMEM((_IDX_MAJOR, _IDX_MINOR), jnp.int32),
        pltpu.VMEM((_IDX_MAJOR, _IDX_MINOR), jnp.int32),
        pltpu.VMEM((_PAIRS_PER_P, _L), jnp.float32),
        pltpu.VMEM((_PAIRS_PER_P, _L), jnp.float32),
        pltpu.VMEM((_ROWS_PER_P, _PS), jnp.float32),
        pltpu.VMEM((_ROWS_PER_P, _PS), jnp.float32),
        pltpu.SemaphoreType.DMA,
        pltpu.SemaphoreType.DMA,
        pltpu.SemaphoreType.DMA,
        pltpu.SemaphoreType.DMA,
    ],
)
def _sc_gather(table, hpos, wpos, out, h_v, w_v, idx_v0, idx_v1,
               pairs_v0, pairs_v1, stage_v0, stage_v1,
               gsem0, gsem1, osem0, osem1):
    wid = lax.axis_index("s") * 2 + lax.axis_index("c")
    sl = pl.ds(wid * (_P_PER_WORKER * _L), _P_PER_WORKER * _L)
    pltpu.sync_copy(hpos.at[sl], h_v)
    pltpu.sync_copy(wpos.at[sl], w_v)
    iota = lax.iota(jnp.int32, _L)

    bufs = ((idx_v0, pairs_v0, stage_v0, gsem0, osem0),
            (idx_v1, pairs_v1, stage_v1, gsem1, osem1))

    def gather_dmas(d, make_only):
        idx_v, pairs_v, _, gsem, _ = bufs[d]
        mk = pltpu.make_async_copy if make_only else pltpu.async_copy
        return [
            mk(
                table.at[idx_v.at[b]],
                pairs_v.at[pl.ds(b * _IDX_MINOR, _IDX_MINOR)],
                gsem,
            )
            for b in range(_IDX_MAJOR)
        ]

    def out_dma(k, d, make_only):
        _, _, stage_v, _, osem = bufs[d]
        p = wid * _P_PER_WORKER + k
        dst = out.at[pl.ds(p * _ROWS_PER_P, _ROWS_PER_P), pl.ds(0, _PS)]
        if make_only:
            return pltpu.make_async_copy(stage_v, dst, osem)
        return pltpu.async_copy(stage_v, dst, osem)

    def gen_and_fire(k, d):
        idx_v = bufs[d][0]
        hv = h_v[pl.ds(k * _L, _L)]
        wv = w_v[pl.ds(k * _L, _L)]
        hh = (hv + iota) & (_H - 1)
        c0 = wv >> 4
        rowbase = hh * _CHUNKS_PER_IMROW + c0
        delta = jnp.where(c0 == _CHUNKS_PER_IMROW - 1,
                          1 - _CHUNKS_PER_IMROW, 1)
        for nc in range(_NC_COMBOS):
            r0 = rowbase + nc * (_H * _CHUNKS_PER_IMROW)
            row = jnp.full((_L,), nc >> 2, jnp.int32)
            col = (nc & 3) * 32 + 2 * iota
            plsc.store_scatter(idx_v, [row, col], r0)
            plsc.store_scatter(idx_v, [row, col + 1], r0 + delta)
        gather_dmas(d, make_only=False)

    def finish(k, d):
        pairs_v, stage_v = bufs[d][1], bufs[d][2]
        for cp in gather_dmas(d, make_only=True):
            cp.wait()
        out_dma(k, d, make_only=True).wait()
        wv = w_v[pl.ds(k * _L, _L)]
        lane_off = (wv & (_L - 1)) + iota
        roff = lane_off >> 4
        coff = lane_off & (_L - 1)

        def per_row(t, c2):
            g = plsc.load_gather(pairs_v, [2 * t + roff, coff])
            stage_v[t] = g
            return c2

        lax.fori_loop(0, _ROWS_PER_P, per_row, 0, unroll=8)
        out_dma(k, d, make_only=False)

    out_dma(0, 0, make_only=False)
    out_dma(1, 1, make_only=False)
    gen_and_fire(0, 0)

    def step(g, carry):
        k0 = 2 * g
        gen_and_fire(k0 + 1, 1)
        finish(k0, 0)
        gen_and_fire(jnp.minimum(k0 + 2, _P_PER_WORKER - 1), 0)
        finish(k0 + 1, 1)
        return carry

    lax.fori_loop(0, _P_PER_WORKER // 2, step, 0)
    for cp in gather_dmas(0, make_only=True):
        cp.wait()
    out_dma(0, 0, make_only=True).wait()
    out_dma(0, 1, make_only=True).wait()


_BP = 16


def _inds_body(h_sref, w_sref, o_ref):
    pid = pl.program_id(0)
    shape = (1, _N, _C, _PS, _PS)
    nn = lax.broadcasted_iota(jnp.int32, shape, 1)
    cc = lax.broadcasted_iota(jnp.int32, shape, 2)
    ii = lax.broadcasted_iota(jnp.int32, shape, 3)
    jj = lax.broadcasted_iota(jnp.int32, shape, 4)
    base = _W * _H * cc + _W * _H * _C * nn
    for b in range(_BP):
        h = h_sref[pid * _BP + b]
        w = w_sref[pid * _BP + b]
        o_ref[pl.ds(b, 1)] = (((w + jj) & (_W - 1))
                              + _W * ((h + ii) & (_H - 1)) + base)


def _tc_inds(h, w):
    grid_spec = pltpu.PrefetchScalarGridSpec(
        num_scalar_prefetch=2,
        grid=(_P // _BP,),
        in_specs=[],
        out_specs=pl.BlockSpec(
            (_BP, _N, _C, _PS, _PS), lambda i, h, w: (i, 0, 0, 0, 0)
        ),
    )
    return pl.pallas_call(
        _inds_body,
        grid_spec=grid_spec,
        out_shape=jax.ShapeDtypeStruct((_P, _N, _C, _PS, _PS), jnp.int32),
    )(h, w)


def kernel(imgs, position_inds_height, position_inds_width):
    h = position_inds_height.astype(jnp.int32)
    w = position_inds_width.astype(jnp.int32)
    table = imgs.reshape(_TABLE_ROWS, _L)
    h_rep = jnp.broadcast_to(h[:, None], (_P, _L)).reshape(-1)
    w_rep = jnp.broadcast_to(w[:, None], (_P, _L)).reshape(-1)
    patches = (
        _sc_gather(table, h_rep, w_rep)[:, :_PS]
        .reshape(_P, _N, _C, _PS, _PS)
    )
    linear_inds = _tc_inds(h, w).reshape(-1)
    return patches, linear_inds


# speedup vs baseline: 1.3835x; 1.0090x over previous
import functools

import jax
import jax.numpy as jnp
from jax import lax
from jax.experimental import pallas as pl
from jax.experimental.pallas import tpu as pltpu
from jax.experimental.pallas import tpu_sc as plsc

_H = 512
_W = 512
_C = 3
_PS = 16
_N = 16
_P = 1024

_L = 16
_NC_COMBOS = _N * _C
_ROWS_PER_P = _NC_COMBOS * _PS
_PAIRS_PER_P = 2 * _ROWS_PER_P
_IDX_MINOR = 128
_IDX_MAJOR = _PAIRS_PER_P // _IDX_MINOR
_OUT_PER_P = _ROWS_PER_P * _PS
_CHUNKS_PER_IMROW = _W // _L
_TABLE_ROWS = _N * _C * _H * _CHUNKS_PER_IMROW

_NUM_WORKERS = 32
_P_PER_WORKER = _P // _NUM_WORKERS

_mesh = plsc.VectorSubcoreMesh(core_axis_name="c", subcore_axis_name="s")

@functools.partial(
    pl.kernel,
    mesh=_mesh,
    out_type=jax.ShapeDtypeStruct((_P * _ROWS_PER_P, 128), jnp.float32),
    compiler_params=pltpu.CompilerParams(
        needs_layout_passes=False, use_tc_tiling_on_sc=False
    ),
    scratch_types=[
        pltpu.VMEM((_P_PER_WORKER * _L,), jnp.int32),
        pltpu.VMEM((_P_PER_WORKER * _L,), jnp.int32),
        pltpu.VMEM((_IDX_MAJOR, _IDX_MINOR), jnp.int32),
        pltpu.VMEM((_IDX_MAJOR, _IDX_MINOR), jnp.int32),
        pltpu.VMEM((_PAIRS_PER_P, _L), jnp.float32),
        pltpu.VMEM((_PAIRS_PER_P, _L), jnp.float32),
        pltpu.VMEM((_ROWS_PER_P, _PS), jnp.float32),
        pltpu.VMEM((_ROWS_PER_P, _PS), jnp.float32),
        pltpu.SemaphoreType.DMA,
        pltpu.SemaphoreType.DMA,
        pltpu.SemaphoreType.DMA,
        pltpu.SemaphoreType.DMA,
    ],
)
def _sc_gather(table, hpos, wpos, out, h_v, w_v, idx_v0, idx_v1,
               pairs_v0, pairs_v1, stage_v0, stage_v1,
               gsem0, gsem1, osem0, osem1):
    wid = lax.axis_index("s") * 2 + lax.axis_index("c")
    sl = pl.ds(wid * (_P_PER_WORKER * _L), _P_PER_WORKER * _L)
    pltpu.sync_copy(hpos.at[sl], h_v)
    pltpu.sync_copy(wpos.at[sl], w_v)
    iota = lax.iota(jnp.int32, _L)

    bufs = ((idx_v0, pairs_v0, stage_v0, gsem0, osem0),
            (idx_v1, pairs_v1, stage_v1, gsem1, osem1))

    def gather_dmas(d, make_only):
        idx_v, pairs_v, _, gsem, _ = bufs[d]
        mk = pltpu.make_async_copy if make_only else pltpu.async_copy
        return [
            mk(
                table.at[idx_v.at[b]],
                pairs_v.at[pl.ds(b * _IDX_MINOR, _IDX_MINOR)],
                gsem,
            )
            for b in range(_IDX_MAJOR)
        ]

    def out_dma(k, d, make_only):
        _, _, stage_v, _, osem = bufs[d]
        p = wid * _P_PER_WORKER + k
        dst = out.at[pl.ds(p * _ROWS_PER_P, _ROWS_PER_P), pl.ds(0, _PS)]
        if make_only:
            return pltpu.make_async_copy(stage_v, dst, osem)
        return pltpu.async_copy(stage_v, dst, osem)

    def gen_and_fire(k, d):
        idx_v = bufs[d][0]
        hv = h_v[pl.ds(k * _L, _L)]
        wv = w_v[pl.ds(k * _L, _L)]
        hh = (hv + iota) & (_H - 1)
        c0 = wv >> 4
        rowbase = hh * _CHUNKS_PER_IMROW + c0
        delta = jnp.where(c0 == _CHUNKS_PER_IMROW - 1,
                          1 - _CHUNKS_PER_IMROW, 1)
        for nc in range(_NC_COMBOS):
            r0 = rowbase + nc * (_H * _CHUNKS_PER_IMROW)
            r, c = nc // 8, (nc % 8) * _L
            idx_v[r, pl.ds(c, _L)] = r0
            idx_v[r + 6, pl.ds(c, _L)] = r0 + delta
        gather_dmas(d, make_only=False)

    def finish(k, d):
        pairs_v, stage_v = bufs[d][1], bufs[d][2]
        for cp in gather_dmas(d, make_only=True):
            cp.wait()
        out_dma(k, d, make_only=True).wait()
        wv = w_v[pl.ds(k * _L, _L)]
        lane_off = (wv & (_L - 1)) + iota
        roff = (lane_off >> 4) * _ROWS_PER_P
        coff = lane_off & (_L - 1)

        def per_row(t, c2):
            g = plsc.load_gather(pairs_v, [t + roff, coff])
            stage_v[t] = g
            return c2

        lax.fori_loop(0, _ROWS_PER_P, per_row, 0, unroll=8)
        out_dma(k, d, make_only=False)

    out_dma(0, 0, make_only=False)
    out_dma(1, 1, make_only=False)
    gen_and_fire(0, 0)

    def step(g, carry):
        k0 = 2 * g
        gen_and_fire(k0 + 1, 1)
        finish(k0, 0)
        gen_and_fire(jnp.minimum(k0 + 2, _P_PER_WORKER - 1), 0)
        finish(k0 + 1, 1)
        return carry

    lax.fori_loop(0, _P_PER_WORKER // 2, step, 0)
    for cp in gather_dmas(0, make_only=True):
        cp.wait()
    out_dma(0, 0, make_only=True).wait()
    out_dma(0, 1, make_only=True).wait()


_BP = 16


def _inds_body(h_sref, w_sref, o_ref):
    pid = pl.program_id(0)
    shape = (1, _N, _C, _PS, _PS)
    nn = lax.broadcasted_iota(jnp.int32, shape, 1)
    cc = lax.broadcasted_iota(jnp.int32, shape, 2)
    ii = lax.broadcasted_iota(jnp.int32, shape, 3)
    jj = lax.broadcasted_iota(jnp.int32, shape, 4)
    base = _W * _H * cc + _W * _H * _C * nn
    for b in range(_BP):
        h = h_sref[pid * _BP + b]
        w = w_sref[pid * _BP + b]
        o_ref[pl.ds(b, 1)] = (((w + jj) & (_W - 1))
                              + _W * ((h + ii) & (_H - 1)) + base)


def _tc_inds(h, w):
    grid_spec = pltpu.PrefetchScalarGridSpec(
        num_scalar_prefetch=2,
        grid=(_P // _BP,),
        in_specs=[],
        out_specs=pl.BlockSpec(
            (_BP, _N, _C, _PS, _PS), lambda i, h, w: (i, 0, 0, 0, 0)
        ),
    )
    return pl.pallas_call(
        _inds_body,
        grid_spec=grid_spec,
        out_shape=jax.ShapeDtypeStruct((_P, _N, _C, _PS, _PS), jnp.int32),
    )(h, w)


def kernel(imgs, position_inds_height, position_inds_width):
    h = position_inds_height.astype(jnp.int32)
    w = position_inds_width.astype(jnp.int32)
    table = imgs.reshape(_TABLE_ROWS, _L)
    h_rep = jnp.broadcast_to(h[:, None], (_P, _L)).reshape(-1)
    w_rep = jnp.broadcast_to(w[:, None], (_P, _L)).reshape(-1)
    patches = (
        _sc_gather(table, h_rep, w_rep)[:, :_PS]
        .reshape(_P, _N, _C, _PS, _PS)
    )
    linear_inds = _tc_inds(h, w).reshape(-1)
    return patches, linear_inds
